# baseline (speedup 1.0000x reference)
import functools

import jax
import jax.numpy as jnp
from jax import lax
from jax.experimental import pallas as pl
from jax.experimental.pallas import tpu as pltpu
from jax.experimental.pallas import tpu_sc as plsc

B, C, H, W, K = 32, 4, 256, 256, 500
HW = H * W
KP = 512
NJ = KP * C
NCHUNK = NJ // 16
NSLAB = B * C

_NC = 2
_NS = 16
NW = _NC * _NS


def _tc_prep(f1, f2, ind, mask, target):
    def k(ind_ref, mask_ref, tgt_ref, f1_ref, f2_ref,
          fsum_ref, idx_ref, mf_ref, tgtf_ref):
        i = pl.program_id(0)
        s = f1_ref[0, 0] + f2_ref[0, 0]
        fsum_ref[...] = s.reshape(H * 128)

        @pl.when(i == 0)
        def _():
            base = lax.broadcasted_iota(jnp.int32, (B, K), 0) * (C * HW)
            indv = ind_ref[...]
            pos = (base
                   + lax.bitwise_and(lax.shift_right_logical(indv, 7), 1)
                   * (H * 128)
                   + lax.shift_right_logical(indv, 8) * 128
                   + lax.bitwise_and(indv, 127))
            for c in range(C):
                idx_ref[:, c * KP:c * KP + K] = pos + c * HW
                idx_ref[:, c * KP + K:(c + 1) * KP] = jnp.zeros((B, KP - K), jnp.int32)
                tgtf_ref[:, c * KP:c * KP + K] = tgt_ref[:, :, c]
                tgtf_ref[:, c * KP + K:(c + 1) * KP] = jnp.zeros((B, KP - K), jnp.float32)
            mf_ref[:, :K] = mask_ref[...].astype(jnp.float32)
            mf_ref[:, K:] = jnp.zeros((B, KP - K), jnp.float32)

    return pl.pallas_call(
        k,
        grid=(NSLAB * 2,),
        in_specs=[
            pl.BlockSpec((B, K), lambda i: (0, 0)),
            pl.BlockSpec((B, K), lambda i: (0, 0)),
            pl.BlockSpec((B, K, C), lambda i: (0, 0, 0)),
            pl.BlockSpec((1, 1, H, 128),
                         lambda i: ((i // 2) // C, (i // 2) % C, 0, i % 2)),
            pl.BlockSpec((1, 1, H, 128),
                         lambda i: ((i // 2) // C, (i // 2) % C, 0, i % 2)),
        ],
        out_specs=[
            pl.BlockSpec((H * 128,), lambda i: (i,)),
            pl.BlockSpec((B, NJ), lambda i: (0, 0)),
            pl.BlockSpec((B, KP), lambda i: (0, 0)),
            pl.BlockSpec((B, NJ), lambda i: (0, 0)),
        ],
        out_shape=[
            jax.ShapeDtypeStruct((NSLAB * HW,), jnp.float32),
            jax.ShapeDtypeStruct((B, NJ), jnp.int32),
            jax.ShapeDtypeStruct((B, KP), jnp.float32),
            jax.ShapeDtypeStruct((B, NJ), jnp.float32),
        ],
    )(ind, mask, target, f1, f2)


def _sc_partials(fsum, idx_all, mask_f, tgt):
    mesh = plsc.VectorSubcoreMesh(core_axis_name="c", subcore_axis_name="s")

    @functools.partial(
        pl.kernel,
        mesh=mesh,
        out_type=[
            jax.ShapeDtypeStruct((NW, 16), jnp.float32),
            jax.ShapeDtypeStruct((NW, 16), jnp.float32),
        ],
        scratch_types=[
            pltpu.VMEM((NJ,), jnp.int32),
            pltpu.VMEM((KP,), jnp.float32),
            pltpu.VMEM((NJ,), jnp.float32),
            pltpu.VMEM((NJ,), jnp.float32),
            pltpu.VMEM((16,), jnp.float32),
            pltpu.VMEM((16,), jnp.float32),
            pltpu.SemaphoreType.DMA,
            pltpu.SemaphoreType.DMA,
            pltpu.SemaphoreType.DMA,
            pltpu.SemaphoreType.DMA,
        ],
    )
    def k(f_hbm, idx_hbm, mask_hbm, tgt_hbm, acc_out, ms_out,
          idx_v, mask_v, tgt_v, p_v, accv, msv,
          semi, semm, semt, semg):
        wid = lax.axis_index("s") * _NC + lax.axis_index("c")
        b = wid

        cpi = pltpu.async_copy(idx_hbm.at[b], idx_v, semi)
        cpm = pltpu.async_copy(mask_hbm.at[b], mask_v, semm)
        cpt = pltpu.async_copy(tgt_hbm.at[b], tgt_v, semt)
        cpi.wait()
        cpg = pltpu.async_copy(f_hbm.at[idx_v], p_v, semg)
        cpm.wait()
        cpt.wait()
        cpg.wait()

        def comp(t, carry):
            acc, ms = carry
            m = mask_v[pl.ds(lax.rem(t, KP // 16) * 16, 16)]
            sl = pl.ds(t * 16, 16)
            e = p_v[sl] - tgt_v[sl]
            return acc + (m * e) * e, ms + m

        zero = jnp.zeros((16,), jnp.float32)
        acc, ms = lax.fori_loop(0, NCHUNK, comp, (zero, zero))
        accv[:] = acc
        msv[:] = ms
        pltpu.sync_copy(accv, acc_out.at[b])
        pltpu.sync_copy(msv, ms_out.at[b])

    return k(fsum, idx_all, mask_f, tgt)


def _tc_reduce(acc, ms):
    def k(acc_ref, ms_ref, out_ref):
        s1 = jnp.sum(acc_ref[...])
        s2 = jnp.sum(ms_ref[...])
        out_ref[0] = s1 / (s2 + 0.0001)

    return pl.pallas_call(
        k,
        out_shape=jax.ShapeDtypeStruct((1,), jnp.float32),
        out_specs=pl.BlockSpec(memory_space=pltpu.SMEM),
    )(acc, ms)


def kernel(output_stage_one, output_stage_two, mask, ind, target):
    fsum, idx_all, mask_f, tgt_flat = _tc_prep(
        output_stage_one, output_stage_two,
        ind.astype(jnp.int32), mask, target)
    acc, ms = _sc_partials(fsum, idx_all, mask_f, tgt_flat)
    return _tc_reduce(acc, ms)[0]


# speedup vs baseline: 1.4818x; 1.4818x over previous
import functools

import jax
import jax.numpy as jnp
from jax import lax
from jax.experimental import pallas as pl
from jax.experimental.pallas import tpu as pltpu
from jax.experimental.pallas import tpu_sc as plsc

B, C, H, W, K = 32, 4, 256, 256, 500
HW = H * W
KP = 512
NJ = KP * C
NCHUNK = NJ // 16
NSLAB = B * C

_NC = 2
_NS = 16
NW = _NC * _NS


def _tc_prep(f1, f2, ind, mask, target):
    def kd(f1_ref, f2_ref, fsum_ref):
        s = f1_ref[0, 0] + f2_ref[0, 0]
        fsum_ref[pl.ds(0, H * 128)] = s[:, :128].reshape(H * 128)
        fsum_ref[pl.ds(H * 128, H * 128)] = s[:, 128:].reshape(H * 128)

    fsum = pl.pallas_call(
        kd,
        grid=(NSLAB,),
        in_specs=[
            pl.BlockSpec((1, 1, H, W), lambda i: (i // C, i % C, 0, 0)),
            pl.BlockSpec((1, 1, H, W), lambda i: (i // C, i % C, 0, 0)),
        ],
        out_specs=pl.BlockSpec((HW,), lambda i: (i,)),
        out_shape=jax.ShapeDtypeStruct((NSLAB * HW,), jnp.float32),
    )(f1, f2)

    def ks(ind_ref, mask_ref, tgt_ref, idx_ref, mf_ref, tgtf_ref):
        base = lax.broadcasted_iota(jnp.int32, (B, K), 0) * (C * HW)
        indv = ind_ref[...]
        pos = (base
               + lax.bitwise_and(lax.shift_right_logical(indv, 7), 1)
               * (H * 128)
               + lax.shift_right_logical(indv, 8) * 128
               + lax.bitwise_and(indv, 127))
        for c in range(C):
            idx_ref[:, c * KP:c * KP + K] = pos + c * HW
            idx_ref[:, c * KP + K:(c + 1) * KP] = jnp.zeros(
                (B, KP - K), jnp.int32)
            tgtf_ref[:, c * KP:c * KP + K] = tgt_ref[:, :, c]
            tgtf_ref[:, c * KP + K:(c + 1) * KP] = jnp.zeros(
                (B, KP - K), jnp.float32)
        mf_ref[:, :K] = mask_ref[...].astype(jnp.float32)
        mf_ref[:, K:] = jnp.zeros((B, KP - K), jnp.float32)

    idx_all, mask_f, tgt_flat = pl.pallas_call(
        ks,
        out_shape=[
            jax.ShapeDtypeStruct((B, NJ), jnp.int32),
            jax.ShapeDtypeStruct((B, KP), jnp.float32),
            jax.ShapeDtypeStruct((B, NJ), jnp.float32),
        ],
    )(ind, mask, target)

    return fsum, idx_all, mask_f, tgt_flat


def _sc_partials(fsum, idx_all, mask_f, tgt):
    mesh = plsc.VectorSubcoreMesh(core_axis_name="c", subcore_axis_name="s")

    @functools.partial(
        pl.kernel,
        mesh=mesh,
        out_type=[
            jax.ShapeDtypeStruct((NW, 16), jnp.float32),
            jax.ShapeDtypeStruct((NW, 16), jnp.float32),
        ],
        scratch_types=[
            pltpu.VMEM((NJ,), jnp.int32),
            pltpu.VMEM((KP,), jnp.float32),
            pltpu.VMEM((NJ,), jnp.float32),
            pltpu.VMEM((NJ,), jnp.float32),
            pltpu.VMEM((16,), jnp.float32),
            pltpu.VMEM((16,), jnp.float32),
            pltpu.SemaphoreType.DMA,
            pltpu.SemaphoreType.DMA,
            pltpu.SemaphoreType.DMA,
            pltpu.SemaphoreType.DMA,
        ],
    )
    def k(f_hbm, idx_hbm, mask_hbm, tgt_hbm, acc_out, ms_out,
          idx_v, mask_v, tgt_v, p_v, accv, msv,
          semi, semm, semt, semg):
        wid = lax.axis_index("s") * _NC + lax.axis_index("c")
        b = wid

        cpi = pltpu.async_copy(idx_hbm.at[b], idx_v, semi)
        cpm = pltpu.async_copy(mask_hbm.at[b], mask_v, semm)
        cpt = pltpu.async_copy(tgt_hbm.at[b], tgt_v, semt)
        cpi.wait()
        cpg = pltpu.async_copy(f_hbm.at[idx_v], p_v, semg)
        cpm.wait()
        cpt.wait()
        cpg.wait()

        def comp(t, carry):
            acc, ms = carry
            m = mask_v[pl.ds(lax.rem(t, KP // 16) * 16, 16)]
            sl = pl.ds(t * 16, 16)
            e = p_v[sl] - tgt_v[sl]
            return acc + (m * e) * e, ms + m

        zero = jnp.zeros((16,), jnp.float32)
        acc, ms = lax.fori_loop(0, NCHUNK, comp, (zero, zero))
        accv[:] = acc
        msv[:] = ms
        pltpu.sync_copy(accv, acc_out.at[b])
        pltpu.sync_copy(msv, ms_out.at[b])

    return k(fsum, idx_all, mask_f, tgt)


def _tc_reduce(acc, ms):
    def k(acc_ref, ms_ref, out_ref):
        s1 = jnp.sum(acc_ref[...])
        s2 = jnp.sum(ms_ref[...])
        out_ref[0] = s1 / (s2 + 0.0001)

    return pl.pallas_call(
        k,
        out_shape=jax.ShapeDtypeStruct((1,), jnp.float32),
        out_specs=pl.BlockSpec(memory_space=pltpu.SMEM),
    )(acc, ms)


def kernel(output_stage_one, output_stage_two, mask, ind, target):
    fsum, idx_all, mask_f, tgt_flat = _tc_prep(
        output_stage_one, output_stage_two,
        ind.astype(jnp.int32), mask, target)
    acc, ms = _sc_partials(fsum, idx_all, mask_f, tgt_flat)
    return _tc_reduce(acc, ms)[0]


# speedup vs baseline: 2.4768x; 1.6715x over previous
import functools

import jax
import jax.numpy as jnp
from jax import lax
from jax.experimental import pallas as pl
from jax.experimental.pallas import tpu as pltpu
from jax.experimental.pallas import tpu_sc as plsc

B, C, H, W, K = 32, 4, 256, 256, 500
HW = H * W
KP = 512
NJ = KP * C
NCHUNK = NJ // 16
NSLAB = B * C
BLK_B = 4

_NC = 2
_NS = 16
NW = _NC * _NS


def _tc_prep(f1, f2, ind, mask, target):
    half = BLK_B * C * H * 128

    def kd(f1_ref, f2_ref, fsum_ref):
        s = f1_ref[...] + f2_ref[...]
        fsum_ref[pl.ds(0, half)] = s[:, :, :, :128].reshape(half)
        fsum_ref[pl.ds(half, half)] = s[:, :, :, 128:].reshape(half)

    fsum = pl.pallas_call(
        kd,
        grid=(B // BLK_B,),
        in_specs=[
            pl.BlockSpec((BLK_B, C, H, W), lambda i: (i, 0, 0, 0)),
            pl.BlockSpec((BLK_B, C, H, W), lambda i: (i, 0, 0, 0)),
        ],
        out_specs=pl.BlockSpec((2 * half,), lambda i: (i,)),
        out_shape=jax.ShapeDtypeStruct((NSLAB * HW,), jnp.float32),
    )(f1, f2)

    def ks(ind_ref, mask_ref, tgt_ref, idx_ref, mf_ref, tgtf_ref):
        bio = lax.broadcasted_iota(jnp.int32, (B, K), 0)
        indv = ind_ref[...]
        plane = H * 128
        pos = ((bio // BLK_B) * (BLK_B * C * HW)
               + lax.bitwise_and(lax.shift_right_logical(indv, 7), 1)
               * (BLK_B * C * plane)
               + (bio % BLK_B) * (C * plane)
               + lax.shift_right_logical(indv, 8) * 128
               + lax.bitwise_and(indv, 127))
        for c in range(C):
            idx_ref[:, c * KP:c * KP + K] = pos + c * plane
            idx_ref[:, c * KP + K:(c + 1) * KP] = jnp.zeros(
                (B, KP - K), jnp.int32)
            tgtf_ref[:, c * KP:c * KP + K] = tgt_ref[:, :, c]
            tgtf_ref[:, c * KP + K:(c + 1) * KP] = jnp.zeros(
                (B, KP - K), jnp.float32)
        mf_ref[:, :K] = mask_ref[...].astype(jnp.float32)
        mf_ref[:, K:] = jnp.zeros((B, KP - K), jnp.float32)

    idx_all, mask_f, tgt_flat = pl.pallas_call(
        ks,
        out_shape=[
            jax.ShapeDtypeStruct((B, NJ), jnp.int32),
            jax.ShapeDtypeStruct((B, KP), jnp.float32),
            jax.ShapeDtypeStruct((B, NJ), jnp.float32),
        ],
    )(ind, mask, target)

    return fsum, idx_all, mask_f, tgt_flat


def _sc_partials(fsum, idx_all, mask_f, tgt):
    mesh = plsc.VectorSubcoreMesh(core_axis_name="c", subcore_axis_name="s")

    @functools.partial(
        pl.kernel,
        mesh=mesh,
        out_type=[
            jax.ShapeDtypeStruct((NW, 16), jnp.float32),
            jax.ShapeDtypeStruct((NW, 16), jnp.float32),
        ],
        scratch_types=[
            pltpu.VMEM((NJ,), jnp.int32),
            pltpu.VMEM((KP,), jnp.float32),
            pltpu.VMEM((NJ,), jnp.float32),
            pltpu.VMEM((NJ,), jnp.float32),
            pltpu.VMEM((16,), jnp.float32),
            pltpu.VMEM((16,), jnp.float32),
            pltpu.SemaphoreType.DMA,
            pltpu.SemaphoreType.DMA,
            pltpu.SemaphoreType.DMA,
            pltpu.SemaphoreType.DMA,
        ],
    )
    def k(f_hbm, idx_hbm, mask_hbm, tgt_hbm, acc_out, ms_out,
          idx_v, mask_v, tgt_v, p_v, accv, msv,
          semi, semm, semt, semg):
        wid = lax.axis_index("s") * _NC + lax.axis_index("c")
        b = wid

        cpi = pltpu.async_copy(idx_hbm.at[b], idx_v, semi)
        cpm = pltpu.async_copy(mask_hbm.at[b], mask_v, semm)
        cpt = pltpu.async_copy(tgt_hbm.at[b], tgt_v, semt)
        cpi.wait()
        cpg = pltpu.async_copy(f_hbm.at[idx_v], p_v, semg)
        cpm.wait()
        cpt.wait()
        cpg.wait()

        def comp(t, carry):
            acc, ms = carry
            m = mask_v[pl.ds(lax.rem(t, KP // 16) * 16, 16)]
            sl = pl.ds(t * 16, 16)
            e = p_v[sl] - tgt_v[sl]
            return acc + (m * e) * e, ms + m

        zero = jnp.zeros((16,), jnp.float32)
        acc, ms = lax.fori_loop(0, NCHUNK, comp, (zero, zero))
        accv[:] = acc
        msv[:] = ms
        pltpu.sync_copy(accv, acc_out.at[b])
        pltpu.sync_copy(msv, ms_out.at[b])

    return k(fsum, idx_all, mask_f, tgt)


def _tc_reduce(acc, ms):
    def k(acc_ref, ms_ref, out_ref):
        s1 = jnp.sum(acc_ref[...])
        s2 = jnp.sum(ms_ref[...])
        out_ref[0] = s1 / (s2 + 0.0001)

    return pl.pallas_call(
        k,
        out_shape=jax.ShapeDtypeStruct((1,), jnp.float32),
        out_specs=pl.BlockSpec(memory_space=pltpu.SMEM),
    )(acc, ms)


def kernel(output_stage_one, output_stage_two, mask, ind, target):
    fsum, idx_all, mask_f, tgt_flat = _tc_prep(
        output_stage_one, output_stage_two,
        ind.astype(jnp.int32), mask, target)
    acc, ms = _sc_partials(fsum, idx_all, mask_f, tgt_flat)
    return _tc_reduce(acc, ms)[0]
